# Initial kernel scaffold; baseline (speedup 1.0000x reference)
import jax
import jax.numpy as jnp
from jax.experimental import pallas as pl


def kernel(hidden_states, router_weight, gate_up_proj, down_proj, shared_gate_proj, shared_up_proj, shared_down_proj, shared_expert_gate):
    raise NotImplementedError("write your pallas kernel here")


# speedup vs baseline: 1.5650x; 1.5650x over previous
import functools

import jax
import jax.numpy as jnp
from jax.experimental import pallas as pl
from jax.experimental.pallas import tpu as pltpu

NE = 8
HID = 1024
MI = 512
SI = 1024
TOKENS = 2048
TM = 256


def _fdot(a, b):
    return jax.lax.dot_general(a, b, (((1,), (1,)), ((), ())),
                               preferred_element_type=jnp.float32)


def _moe_body(x_ref, rw_ref, gu_ref, dn_ref, sg_ref, su_ref, sd_ref, seg_ref,
              out_ref):
    x32 = x_ref[...]
    xb = x32.astype(jnp.bfloat16)

    logits = jax.lax.dot_general(
        x32, rw_ref[...], (((1,), (1,)), ((), ())),
        preferred_element_type=jnp.float32)
    probs = jax.nn.softmax(logits, axis=-1)
    iota8 = jax.lax.broadcasted_iota(jnp.int32, (TM, NE), 1)
    v1 = jnp.max(probs, axis=1, keepdims=True)
    i1 = jnp.min(jnp.where(probs >= v1, iota8, NE), axis=1, keepdims=True)
    pm = jnp.where(iota8 == i1, -1.0, probs)
    v2 = jnp.max(pm, axis=1, keepdims=True)
    i2 = jnp.min(jnp.where(pm >= v2, iota8, NE), axis=1, keepdims=True)
    rs = v1 + v2
    wd = (jnp.where(iota8 == i1, v1, 0.0)
          + jnp.where(iota8 == i2, v2, 0.0)) / rs

    g = _fdot(xb, sg_ref[...])
    u = _fdot(xb, su_ref[...])
    hs = (jax.nn.silu(g) * u).astype(jnp.bfloat16)
    sy = _fdot(hs, sd_ref[...])
    gate = jax.nn.sigmoid(jax.lax.dot_general(
        x32, seg_ref[...], (((1,), (1,)), ((), ())),
        preferred_element_type=jnp.float32,
        precision=jax.lax.Precision.HIGHEST))
    acc = gate * sy

    for e in range(NE):
        gu = _fdot(xb, gu_ref[e])
        h = (jax.nn.silu(gu[:, :MI]) * gu[:, MI:]).astype(jnp.bfloat16)
        ye = _fdot(h, dn_ref[e])
        acc = acc + wd[:, e:e + 1] * ye

    out_ref[...] = acc


@jax.jit
def _moe(x, rw, gu_b, dn_b, sg_b, su_b, sd_b, seg):
    grid = (TOKENS // TM,)
    full = lambda shape: pl.BlockSpec(shape, lambda t: tuple(0 for _ in shape))
    return pl.pallas_call(
        _moe_body,
        grid=grid,
        in_specs=[
            pl.BlockSpec((TM, HID), lambda t: (t, 0)),
            full((NE, HID)),
            full((NE, 2 * MI, HID)),
            full((NE, HID, MI)),
            full((SI, HID)),
            full((SI, HID)),
            full((HID, SI)),
            full((1, HID)),
        ],
        out_specs=pl.BlockSpec((TM, HID), lambda t: (t, 0)),
        out_shape=jax.ShapeDtypeStruct((TOKENS, HID), jnp.float32),
    )(x, rw, gu_b, dn_b, sg_b, su_b, sd_b, seg)


def kernel(hidden_states, router_weight, gate_up_proj, down_proj,
           shared_gate_proj, shared_up_proj, shared_down_proj,
           shared_expert_gate):
    B, S, H = hidden_states.shape
    x = hidden_states.reshape(-1, H)
    out = _moe(x,
               router_weight,
               gate_up_proj.astype(jnp.bfloat16),
               down_proj.astype(jnp.bfloat16),
               shared_gate_proj.astype(jnp.bfloat16),
               shared_up_proj.astype(jnp.bfloat16),
               shared_down_proj.astype(jnp.bfloat16),
               shared_expert_gate)
    return out.reshape(B, S, H)
